# Initial kernel scaffold; baseline (speedup 1.0000x reference)
import jax
import jax.numpy as jnp
from jax.experimental import pallas as pl


def kernel(gene, cna, mutation, feature_drug, Wg, bg, Wc, bc, Wm, bm, Wd, bd, weights3, attW1, attb1, attW2, attb2, cW1, cb1, cW2, cb2, dW1, db1, dW2, db2, kcW, kcb, kdW, kdb):
    raise NotImplementedError("write your pallas kernel here")


# speedup vs baseline: 3.9791x; 3.9791x over previous
import functools

import jax
import jax.numpy as jnp
from jax.experimental import pallas as pl
from jax.experimental.pallas import tpu as pltpu

_SIGMA = 23.0
_TOPK = 10
_HI = jax.lax.Precision.HIGHEST


def _zn_body(g_ref, c_ref, go_ref, co_ref):
    for src, dst in ((g_ref, go_ref), (c_ref, co_ref)):
        x = src[...]
        mu = jnp.mean(x, axis=0, keepdims=True)
        sd = jnp.sqrt(jnp.mean((x - mu) ** 2, axis=0, keepdims=True))
        dst[...] = (x - mu) / (sd + 1e-8)


def _z_normalize(gene, cna):
    return pl.pallas_call(
        _zn_body,
        out_shape=[jax.ShapeDtypeStruct(gene.shape, jnp.float32),
                   jax.ShapeDtypeStruct(cna.shape, jnp.float32)],
    )(gene, cna)


def _kernels_body(gr_ref, gc_ref, cr_ref, cc_ref, mr_ref, mc_ref,
                  wg_ref, bg_ref, wc_ref, bc_ref, wm_ref, bm_ref,
                  kg_ref, kc_ref, km_ref, sh_ref, nrm_ref):
    i = pl.program_id(0)
    j = pl.program_id(1)

    @pl.when((i == 0) & (j == 0))
    def _():
        nrm_ref[0] = 0.0
        nrm_ref[1] = 0.0
        nrm_ref[2] = 0.0

    gr = gr_ref[...]
    gc = gc_ref[...]
    dims = (((1,), (1,)), ((), ()))
    gram_g = jax.lax.dot_general(gr, gc, dims, precision=_HI)
    sq_r = jnp.sum(gr * gr, axis=1, keepdims=True)
    sq_c = jnp.sum(gc * gc, axis=1, keepdims=True)
    d2 = jnp.maximum(sq_r + sq_c.T - 2.0 * gram_g, 0.0)
    kg = jnp.exp(d2 * (-1.0 / (2.0 * _SIGMA * _SIGMA)))
    kg_ref[...] = kg

    cr = cr_ref[...]
    cc = cc_ref[...]
    p = jax.lax.dot_general(cr, cc, dims, precision=_HI) + 1.0
    kc = p * p * p
    kc_ref[...] = kc

    mr = mr_ref[...]
    mc = mc_ref[...]
    inter = jax.lax.dot_general(mr, mc, dims, precision=_HI)
    s_r = jnp.sum(mr, axis=1, keepdims=True)
    s_c = jnp.sum(mc, axis=1, keepdims=True)
    km = inter / (s_r + s_c.T - inter + 1e-8)
    km_ref[...] = km

    nrm_ref[0] += jnp.sum(kg * kg)
    nrm_ref[1] += jnp.sum(kc * kc)
    nrm_ref[2] += jnp.sum(km * km)

    @pl.when(j == 0)
    def _():
        ge = jnp.dot(gr, wg_ref[...], precision=_HI) + bg_ref[...]
        ce = jnp.dot(cr, wc_ref[...], precision=_HI) + bc_ref[...]
        me = jnp.dot(mr, wm_ref[...], precision=_HI) + bm_ref[...]
        sh_ref[...] = (ge + ce + me) * (1.0 / 3.0)


def _cell_kernels(gene_n, cna_n, mutation, Wg, bg, Wc, bc, Wm, bm, tile=512):
    n = gene_n.shape[0]
    grid = (n // tile, n // tile)
    f = gene_n.shape[1]
    sd = Wg.shape[1]
    row = pl.BlockSpec((tile, f), lambda i, j: (i, 0))
    col = pl.BlockSpec((tile, f), lambda i, j: (j, 0))
    wsp = pl.BlockSpec((f, sd), lambda i, j: (0, 0))
    bsp = pl.BlockSpec((1, sd), lambda i, j: (0, 0))
    out = pl.BlockSpec((tile, tile), lambda i, j: (i, j))
    shsp = pl.BlockSpec((tile, sd), lambda i, j: (i, 0))
    return pl.pallas_call(
        _kernels_body,
        grid=grid,
        in_specs=[row, col, row, col, row, col, wsp, bsp, wsp, bsp, wsp, bsp],
        out_specs=[out, out, out, shsp,
                   pl.BlockSpec(memory_space=pltpu.SMEM)],
        out_shape=[jax.ShapeDtypeStruct((n, n), jnp.float32),
                   jax.ShapeDtypeStruct((n, n), jnp.float32),
                   jax.ShapeDtypeStruct((n, n), jnp.float32),
                   jax.ShapeDtypeStruct((n, sd), jnp.float32),
                   jax.ShapeDtypeStruct((3,), jnp.float32)],
    )(gene_n, gene_n, cna_n, cna_n, mutation, mutation,
      Wg, bg.reshape(1, -1), Wc, bc.reshape(1, -1), Wm, bm.reshape(1, -1))


def _mlp_body(sh_ref, fd_ref, aw1_ref, ab1_ref, aw2_ref, ab2_ref,
              cw1_ref, cb1_ref, cw2_ref, cb2_ref,
              wd_ref, bd_ref, dw1_ref, db1_ref, dw2_ref, db2_ref,
              dyn_ref, ac_ref, ad_ref):
    sh = sh_ref[...]
    h = jnp.maximum(jnp.dot(sh, aw1_ref[...], precision=_HI) + ab1_ref[...], 0.0)
    logits = jnp.dot(h, aw2_ref[...], precision=_HI) + ab2_ref[...]
    mx = jnp.max(logits, axis=1, keepdims=True)
    e = jnp.exp(logits - mx)
    dyn = e / jnp.sum(e, axis=1, keepdims=True)
    dyn_ref[...] = jnp.sum(dyn, axis=0, keepdims=True)

    hc = jnp.maximum(jnp.dot(sh, cw1_ref[...], precision=_HI) + cb1_ref[...], 0.0)
    sc = jax.nn.sigmoid(jnp.dot(hc, cw2_ref[...], precision=_HI) + cb2_ref[...])
    ac_ref[...] = sc / (jnp.sum(sc) + 1e-8)

    de = jnp.dot(fd_ref[...], wd_ref[...], precision=_HI) + bd_ref[...]
    hd = jnp.maximum(jnp.dot(de, dw1_ref[...], precision=_HI) + db1_ref[...], 0.0)
    sd_ = jax.nn.sigmoid(jnp.dot(hd, dw2_ref[...], precision=_HI) + db2_ref[...])
    ad_ref[...] = sd_ / (jnp.sum(sd_) + 1e-8)


def _mlps(shared, feature_drug, attW1, attb1, attW2, attb2,
          cW1, cb1, cW2, cb2, Wd, bd, dW1, db1, dW2, db2):
    n = shared.shape[0]
    nd = feature_drug.shape[0]
    return pl.pallas_call(
        _mlp_body,
        out_shape=[jax.ShapeDtypeStruct((1, 3), jnp.float32),
                   jax.ShapeDtypeStruct((n, 1), jnp.float32),
                   jax.ShapeDtypeStruct((nd, 1), jnp.float32)],
    )(shared, feature_drug, attW1, attb1.reshape(1, -1), attW2,
      attb2.reshape(1, -1), cW1, cb1.reshape(1, -1), cW2, cb2.reshape(1, -1),
      Wd, bd.reshape(1, -1), dW1, db1.reshape(1, -1), dW2, db2.reshape(1, -1))


def _topk_filter(weighted, vals_src, k):
    n, m = weighted.shape
    iota = jax.lax.broadcasted_iota(jnp.int32, (n, m), 1)
    big = jnp.int32(2 ** 30)
    filt = jnp.zeros_like(weighted)
    wk = weighted
    for _ in range(k):
        mx = jnp.max(wk, axis=1, keepdims=True)
        cand = jnp.where(wk >= mx, iota, big)
        amin = jnp.min(cand, axis=1, keepdims=True)
        sel = cand == amin
        filt = filt + jnp.where(sel, vals_src, 0.0)
        wk = jnp.where(sel, -jnp.inf, wk)
    return filt


def _fuse_body(w3_ref, kg_ref, kc_ref, km_ref, att_ref, kcw_ref, kcb_ref,
               out_ref):
    fused = (w3_ref[0] * kg_ref[...] + w3_ref[1] * kc_ref[...]
             + w3_ref[2] * km_ref[...])
    weighted = fused * att_ref[...]
    filt = _topk_filter(weighted, fused, _TOPK)
    out_ref[...] = jnp.dot(filt, kcw_ref[...], precision=_HI) + kcb_ref[...]


def _fuse_topk(w3, kg, kc, km, att_row, kcW, kcb, block=128):
    n = kg.shape[0]
    sd = kcW.shape[1]
    grid = (n // block,)
    ksp = pl.BlockSpec((block, n), lambda i: (i, 0))
    return pl.pallas_call(
        _fuse_body,
        grid=grid,
        in_specs=[pl.BlockSpec(memory_space=pltpu.SMEM),
                  ksp, ksp, ksp,
                  pl.BlockSpec((1, n), lambda i: (0, 0)),
                  pl.BlockSpec((n, sd), lambda i: (0, 0)),
                  pl.BlockSpec((1, sd), lambda i: (0, 0))],
        out_specs=pl.BlockSpec((block, sd), lambda i: (i, 0)),
        out_shape=jax.ShapeDtypeStruct((n, sd), jnp.float32),
    )(w3, kg, kc, km, att_row, kcW, kcb.reshape(1, -1))


def _drug_body(fd_ref, att_ref, kdw_ref, kdb_ref, out_ref):
    fd = fd_ref[...]
    dims = (((1,), (1,)), ((), ()))
    inter = jax.lax.dot_general(fd, fd, dims, precision=_HI)
    s = jnp.sum(fd, axis=1, keepdims=True)
    jac = inter / (s + s.T - inter + 1e-8)
    weighted = jac * att_ref[...]
    filt = _topk_filter(weighted, jac, _TOPK)
    d = jnp.sum(filt, axis=1, keepdims=True)
    dinv = jax.lax.rsqrt(d + 1e-8)
    fn = filt * dinv * dinv.T
    out_ref[...] = jnp.dot(fn, kdw_ref[...], precision=_HI) + kdb_ref[...]


def _drug_branch(feature_drug, att_row, kdW, kdb):
    nd = feature_drug.shape[0]
    sd = kdW.shape[1]
    return pl.pallas_call(
        _drug_body,
        out_shape=jax.ShapeDtypeStruct((nd, sd), jnp.float32),
    )(feature_drug, att_row, kdW, kdb.reshape(1, -1))


@jax.jit
def kernel(gene, cna, mutation, feature_drug, Wg, bg, Wc, bc, Wm, bm, Wd, bd,
           weights3, attW1, attb1, attW2, attb2, cW1, cb1, cW2, cb2,
           dW1, db1, dW2, db2, kcW, kcb, kdW, kdb):
    gene_n, cna_n = _z_normalize(gene, cna)
    kg, kc, km, shared, norm_sq = _cell_kernels(
        gene_n, cna_n, mutation, Wg, bg, Wc, bc, Wm, bm)
    dyn_sum, att_cell, att_drug = _mlps(
        shared, feature_drug, attW1, attb1, attW2, attb2,
        cW1, cb1, cW2, cb2, Wd, bd, dW1, db1, dW2, db2)

    n_cell = gene.shape[0]
    stat = jax.nn.softmax(weights3)
    dyn_mean = dyn_sum[0] / n_cell
    w = 0.5 * stat + 0.5 * dyn_mean
    w3 = w / (jnp.sqrt(norm_sq) + 1e-8)

    cell_feat = _fuse_topk(w3, kg, kc, km,
                           att_cell.reshape(1, -1), kcW, kcb)
    drug_feat = _drug_branch(feature_drug, att_drug.reshape(1, -1), kdW, kdb)
    return jnp.concatenate([cell_feat, drug_feat], axis=0)


# speedup vs baseline: 4.7344x; 1.1898x over previous
import functools

import jax
import jax.numpy as jnp
from jax.experimental import pallas as pl
from jax.experimental.pallas import tpu as pltpu

_SIGMA = 23.0
_TOPK = 10
_HI = jax.lax.Precision.HIGHEST


def _zn_body(g_ref, c_ref, go_ref, co_ref):
    for src, dst in ((g_ref, go_ref), (c_ref, co_ref)):
        x = src[...]
        mu = jnp.mean(x, axis=0, keepdims=True)
        sd = jnp.sqrt(jnp.mean((x - mu) ** 2, axis=0, keepdims=True))
        dst[...] = (x - mu) / (sd + 1e-8)


def _z_normalize(gene, cna):
    return pl.pallas_call(
        _zn_body,
        out_shape=[jax.ShapeDtypeStruct(gene.shape, jnp.float32),
                   jax.ShapeDtypeStruct(cna.shape, jnp.float32)],
    )(gene, cna)


def _kernels_body(gr_ref, gc_ref, cr_ref, cc_ref, mr_ref, mc_ref,
                  wg_ref, bg_ref, wc_ref, bc_ref, wm_ref, bm_ref,
                  kg_ref, kc_ref, km_ref, sh_ref, nrm_ref):
    i = pl.program_id(0)
    j = pl.program_id(1)

    @pl.when((i == 0) & (j == 0))
    def _():
        nrm_ref[0] = 0.0
        nrm_ref[1] = 0.0
        nrm_ref[2] = 0.0

    gr = gr_ref[...]
    gc = gc_ref[...]
    dims = (((1,), (1,)), ((), ()))
    gram_g = jax.lax.dot_general(gr, gc, dims, precision=_HI)
    sq_r = jnp.sum(gr * gr, axis=1, keepdims=True)
    sq_c = jnp.sum(gc * gc, axis=1, keepdims=True)
    d2 = jnp.maximum(sq_r + sq_c.T - 2.0 * gram_g, 0.0)
    kg = jnp.exp(d2 * (-1.0 / (2.0 * _SIGMA * _SIGMA)))
    kg_ref[...] = kg

    cr = cr_ref[...]
    cc = cc_ref[...]
    p = jax.lax.dot_general(cr, cc, dims, precision=_HI) + 1.0
    kc = p * p * p
    kc_ref[...] = kc

    mr = mr_ref[...]
    mc = mc_ref[...]
    inter = jax.lax.dot_general(mr, mc, dims, precision=_HI)
    s_r = jnp.sum(mr, axis=1, keepdims=True)
    s_c = jnp.sum(mc, axis=1, keepdims=True)
    km = inter / (s_r + s_c.T - inter + 1e-8)
    km_ref[...] = km

    nrm_ref[0] += jnp.sum(kg * kg)
    nrm_ref[1] += jnp.sum(kc * kc)
    nrm_ref[2] += jnp.sum(km * km)

    @pl.when(j == 0)
    def _():
        ge = jnp.dot(gr, wg_ref[...], precision=_HI) + bg_ref[...]
        ce = jnp.dot(cr, wc_ref[...], precision=_HI) + bc_ref[...]
        me = jnp.dot(mr, wm_ref[...], precision=_HI) + bm_ref[...]
        sh_ref[...] = (ge + ce + me) * (1.0 / 3.0)


def _cell_kernels(gene_n, cna_n, mutation, Wg, bg, Wc, bc, Wm, bm, tile=512):
    n = gene_n.shape[0]
    grid = (n // tile, n // tile)
    f = gene_n.shape[1]
    sd = Wg.shape[1]
    row = pl.BlockSpec((tile, f), lambda i, j: (i, 0))
    col = pl.BlockSpec((tile, f), lambda i, j: (j, 0))
    wsp = pl.BlockSpec((f, sd), lambda i, j: (0, 0))
    bsp = pl.BlockSpec((1, sd), lambda i, j: (0, 0))
    out = pl.BlockSpec((tile, tile), lambda i, j: (i, j))
    shsp = pl.BlockSpec((tile, sd), lambda i, j: (i, 0))
    return pl.pallas_call(
        _kernels_body,
        grid=grid,
        in_specs=[row, col, row, col, row, col, wsp, bsp, wsp, bsp, wsp, bsp],
        out_specs=[out, out, out, shsp,
                   pl.BlockSpec(memory_space=pltpu.SMEM)],
        out_shape=[jax.ShapeDtypeStruct((n, n), jnp.float32),
                   jax.ShapeDtypeStruct((n, n), jnp.float32),
                   jax.ShapeDtypeStruct((n, n), jnp.float32),
                   jax.ShapeDtypeStruct((n, sd), jnp.float32),
                   jax.ShapeDtypeStruct((3,), jnp.float32)],
    )(gene_n, gene_n, cna_n, cna_n, mutation, mutation,
      Wg, bg.reshape(1, -1), Wc, bc.reshape(1, -1), Wm, bm.reshape(1, -1))


def _mlp_body(sh_ref, fd_ref, aw1_ref, ab1_ref, aw2_ref, ab2_ref,
              cw1_ref, cb1_ref, cw2_ref, cb2_ref,
              wd_ref, bd_ref, dw1_ref, db1_ref, dw2_ref, db2_ref,
              dyn_ref, ac_ref, ad_ref):
    sh = sh_ref[...]
    h = jnp.maximum(jnp.dot(sh, aw1_ref[...], precision=_HI) + ab1_ref[...], 0.0)
    logits = jnp.dot(h, aw2_ref[...], precision=_HI) + ab2_ref[...]
    mx = jnp.max(logits, axis=1, keepdims=True)
    e = jnp.exp(logits - mx)
    dyn = e / jnp.sum(e, axis=1, keepdims=True)
    dyn_ref[...] = jnp.sum(dyn, axis=0, keepdims=True)

    hc = jnp.maximum(jnp.dot(sh, cw1_ref[...], precision=_HI) + cb1_ref[...], 0.0)
    sc = jax.nn.sigmoid(jnp.dot(hc, cw2_ref[...], precision=_HI) + cb2_ref[...])
    ac_ref[...] = sc / (jnp.sum(sc) + 1e-8)

    de = jnp.dot(fd_ref[...], wd_ref[...], precision=_HI) + bd_ref[...]
    hd = jnp.maximum(jnp.dot(de, dw1_ref[...], precision=_HI) + db1_ref[...], 0.0)
    sd_ = jax.nn.sigmoid(jnp.dot(hd, dw2_ref[...], precision=_HI) + db2_ref[...])
    ad_ref[...] = sd_ / (jnp.sum(sd_) + 1e-8)


def _mlps(shared, feature_drug, attW1, attb1, attW2, attb2,
          cW1, cb1, cW2, cb2, Wd, bd, dW1, db1, dW2, db2):
    n = shared.shape[0]
    nd = feature_drug.shape[0]
    return pl.pallas_call(
        _mlp_body,
        out_shape=[jax.ShapeDtypeStruct((1, 3), jnp.float32),
                   jax.ShapeDtypeStruct((n, 1), jnp.float32),
                   jax.ShapeDtypeStruct((nd, 1), jnp.float32)],
    )(shared, feature_drug, attW1, attb1.reshape(1, -1), attW2,
      attb2.reshape(1, -1), cW1, cb1.reshape(1, -1), cW2, cb2.reshape(1, -1),
      Wd, bd.reshape(1, -1), dW1, db1.reshape(1, -1), dW2, db2.reshape(1, -1))


def _topk_filter(weighted, vals_src, k):
    filt = jnp.zeros_like(weighted)
    wk = weighted
    for _ in range(k):
        mx = jnp.max(wk, axis=1, keepdims=True)
        sel = wk >= mx
        filt = filt + jnp.where(sel, vals_src, 0.0)
        wk = jnp.where(sel, -jnp.inf, wk)
    return filt


def _fuse_body(w3_ref, kg_ref, kc_ref, km_ref, att_ref, kcw_ref, kcb_ref,
               out_ref):
    fused = (w3_ref[0] * kg_ref[...] + w3_ref[1] * kc_ref[...]
             + w3_ref[2] * km_ref[...])
    weighted = fused * att_ref[...]
    filt = _topk_filter(weighted, fused, _TOPK)
    out_ref[...] = jnp.dot(filt, kcw_ref[...], precision=_HI) + kcb_ref[...]


def _fuse_topk(w3, kg, kc, km, att_row, kcW, kcb, block=128):
    n = kg.shape[0]
    sd = kcW.shape[1]
    grid = (n // block,)
    ksp = pl.BlockSpec((block, n), lambda i: (i, 0))
    return pl.pallas_call(
        _fuse_body,
        grid=grid,
        in_specs=[pl.BlockSpec(memory_space=pltpu.SMEM),
                  ksp, ksp, ksp,
                  pl.BlockSpec((1, n), lambda i: (0, 0)),
                  pl.BlockSpec((n, sd), lambda i: (0, 0)),
                  pl.BlockSpec((1, sd), lambda i: (0, 0))],
        out_specs=pl.BlockSpec((block, sd), lambda i: (i, 0)),
        out_shape=jax.ShapeDtypeStruct((n, sd), jnp.float32),
    )(w3, kg, kc, km, att_row, kcW, kcb.reshape(1, -1))


def _drug_body(fd_ref, att_ref, kdw_ref, kdb_ref, out_ref):
    fd = fd_ref[...]
    dims = (((1,), (1,)), ((), ()))
    inter = jax.lax.dot_general(fd, fd, dims, precision=_HI)
    s = jnp.sum(fd, axis=1, keepdims=True)
    jac = inter / (s + s.T - inter + 1e-8)
    weighted = jac * att_ref[...]
    filt = _topk_filter(weighted, jac, _TOPK)
    d = jnp.sum(filt, axis=1, keepdims=True)
    dinv = jax.lax.rsqrt(d + 1e-8)
    fn = filt * dinv * dinv.T
    out_ref[...] = jnp.dot(fn, kdw_ref[...], precision=_HI) + kdb_ref[...]


def _drug_branch(feature_drug, att_row, kdW, kdb):
    nd = feature_drug.shape[0]
    sd = kdW.shape[1]
    return pl.pallas_call(
        _drug_body,
        out_shape=jax.ShapeDtypeStruct((nd, sd), jnp.float32),
    )(feature_drug, att_row, kdW, kdb.reshape(1, -1))


@jax.jit
def kernel(gene, cna, mutation, feature_drug, Wg, bg, Wc, bc, Wm, bm, Wd, bd,
           weights3, attW1, attb1, attW2, attb2, cW1, cb1, cW2, cb2,
           dW1, db1, dW2, db2, kcW, kcb, kdW, kdb):
    gene_n, cna_n = _z_normalize(gene, cna)
    kg, kc, km, shared, norm_sq = _cell_kernels(
        gene_n, cna_n, mutation, Wg, bg, Wc, bc, Wm, bm)
    dyn_sum, att_cell, att_drug = _mlps(
        shared, feature_drug, attW1, attb1, attW2, attb2,
        cW1, cb1, cW2, cb2, Wd, bd, dW1, db1, dW2, db2)

    n_cell = gene.shape[0]
    stat = jax.nn.softmax(weights3)
    dyn_mean = dyn_sum[0] / n_cell
    w = 0.5 * stat + 0.5 * dyn_mean
    w3 = w / (jnp.sqrt(norm_sq) + 1e-8)

    cell_feat = _fuse_topk(w3, kg, kc, km,
                           att_cell.reshape(1, -1), kcW, kcb)
    drug_feat = _drug_branch(feature_drug, att_drug.reshape(1, -1), kdW, kdb)
    return jnp.concatenate([cell_feat, drug_feat], axis=0)


# speedup vs baseline: 7.1378x; 1.5077x over previous
import functools

import jax
import jax.numpy as jnp
from jax.experimental import pallas as pl
from jax.experimental.pallas import tpu as pltpu

_SIGMA = 23.0
_TOPK = 10
_HI = jax.lax.Precision.HIGHEST


def _dot3(a, b, dims):
    f = functools.partial(jax.lax.dot_general, dimension_numbers=dims,
                          preferred_element_type=jnp.float32)
    ah = a.astype(jnp.bfloat16)
    al = (a - ah.astype(jnp.float32)).astype(jnp.bfloat16)
    bh = b.astype(jnp.bfloat16)
    bl = (b - bh.astype(jnp.float32)).astype(jnp.bfloat16)
    return f(ah, bh) + (f(ah, bl) + f(al, bh))


def _dot1(a, b, dims):
    return jax.lax.dot_general(a.astype(jnp.bfloat16), b.astype(jnp.bfloat16),
                               dimension_numbers=dims,
                               preferred_element_type=jnp.float32)


def _zn_body(g_ref, c_ref, go_ref, co_ref):
    for src, dst in ((g_ref, go_ref), (c_ref, co_ref)):
        x = src[...]
        mu = jnp.mean(x, axis=0, keepdims=True)
        sd = jnp.sqrt(jnp.mean((x - mu) ** 2, axis=0, keepdims=True))
        dst[...] = (x - mu) / (sd + 1e-8)


def _z_normalize(gene, cna):
    return pl.pallas_call(
        _zn_body,
        out_shape=[jax.ShapeDtypeStruct(gene.shape, jnp.float32),
                   jax.ShapeDtypeStruct(cna.shape, jnp.float32)],
    )(gene, cna)


def _kernels_body(gr_ref, gc_ref, cr_ref, cc_ref, mr_ref, mc_ref,
                  wg_ref, bg_ref, wc_ref, bc_ref, wm_ref, bm_ref,
                  kg_ref, kc_ref, km_ref, sh_ref, nrm_ref):
    i = pl.program_id(0)
    j = pl.program_id(1)

    @pl.when((i == 0) & (j == 0))
    def _():
        nrm_ref[0] = 0.0
        nrm_ref[1] = 0.0
        nrm_ref[2] = 0.0

    gr = gr_ref[...]
    gc = gc_ref[...]
    dims = (((1,), (1,)), ((), ()))
    gram_g = _dot3(gr, gc, dims)
    sq_r = jnp.sum(gr * gr, axis=1, keepdims=True)
    sq_c = jnp.sum(gc * gc, axis=1, keepdims=True)
    d2 = jnp.maximum(sq_r + sq_c.T - 2.0 * gram_g, 0.0)
    kg = jnp.exp(d2 * (-1.0 / (2.0 * _SIGMA * _SIGMA)))
    kg_ref[...] = kg

    cr = cr_ref[...]
    cc = cc_ref[...]
    p = _dot3(cr, cc, dims) + 1.0
    kc = p * p * p
    kc_ref[...] = kc

    mr = mr_ref[...]
    mc = mc_ref[...]
    inter = _dot1(mr, mc, dims)
    s_r = jnp.sum(mr, axis=1, keepdims=True)
    s_c = jnp.sum(mc, axis=1, keepdims=True)
    km = inter / (s_r + s_c.T - inter + 1e-8)
    km_ref[...] = km

    nrm_ref[0] += jnp.sum(kg * kg)
    nrm_ref[1] += jnp.sum(kc * kc)
    nrm_ref[2] += jnp.sum(km * km)

    @pl.when(j == 0)
    def _():
        ge = jnp.dot(gr, wg_ref[...], precision=_HI) + bg_ref[...]
        ce = jnp.dot(cr, wc_ref[...], precision=_HI) + bc_ref[...]
        me = jnp.dot(mr, wm_ref[...], precision=_HI) + bm_ref[...]
        sh_ref[...] = (ge + ce + me) * (1.0 / 3.0)


def _cell_kernels(gene_n, cna_n, mutation, Wg, bg, Wc, bc, Wm, bm, tile=512):
    n = gene_n.shape[0]
    grid = (n // tile, n // tile)
    f = gene_n.shape[1]
    sd = Wg.shape[1]
    row = pl.BlockSpec((tile, f), lambda i, j: (i, 0))
    col = pl.BlockSpec((tile, f), lambda i, j: (j, 0))
    wsp = pl.BlockSpec((f, sd), lambda i, j: (0, 0))
    bsp = pl.BlockSpec((1, sd), lambda i, j: (0, 0))
    out = pl.BlockSpec((tile, tile), lambda i, j: (i, j))
    shsp = pl.BlockSpec((tile, sd), lambda i, j: (i, 0))
    return pl.pallas_call(
        _kernels_body,
        grid=grid,
        in_specs=[row, col, row, col, row, col, wsp, bsp, wsp, bsp, wsp, bsp],
        out_specs=[out, out, out, shsp,
                   pl.BlockSpec(memory_space=pltpu.SMEM)],
        out_shape=[jax.ShapeDtypeStruct((n, n), jnp.float32),
                   jax.ShapeDtypeStruct((n, n), jnp.float32),
                   jax.ShapeDtypeStruct((n, n), jnp.float32),
                   jax.ShapeDtypeStruct((n, sd), jnp.float32),
                   jax.ShapeDtypeStruct((3,), jnp.float32)],
    )(gene_n, gene_n, cna_n, cna_n, mutation, mutation,
      Wg, bg.reshape(1, -1), Wc, bc.reshape(1, -1), Wm, bm.reshape(1, -1))


def _mlp_body(sh_ref, fd_ref, aw1_ref, ab1_ref, aw2_ref, ab2_ref,
              cw1_ref, cb1_ref, cw2_ref, cb2_ref,
              wd_ref, bd_ref, dw1_ref, db1_ref, dw2_ref, db2_ref,
              dyn_ref, ac_ref, ad_ref):
    sh = sh_ref[...]
    h = jnp.maximum(jnp.dot(sh, aw1_ref[...], precision=_HI) + ab1_ref[...], 0.0)
    logits = jnp.dot(h, aw2_ref[...], precision=_HI) + ab2_ref[...]
    mx = jnp.max(logits, axis=1, keepdims=True)
    e = jnp.exp(logits - mx)
    dyn = e / jnp.sum(e, axis=1, keepdims=True)
    dyn_ref[...] = jnp.sum(dyn, axis=0, keepdims=True)

    hc = jnp.maximum(jnp.dot(sh, cw1_ref[...], precision=_HI) + cb1_ref[...], 0.0)
    sc = jax.nn.sigmoid(jnp.dot(hc, cw2_ref[...], precision=_HI) + cb2_ref[...])
    ac_ref[...] = sc / (jnp.sum(sc) + 1e-8)

    de = jnp.dot(fd_ref[...], wd_ref[...], precision=_HI) + bd_ref[...]
    hd = jnp.maximum(jnp.dot(de, dw1_ref[...], precision=_HI) + db1_ref[...], 0.0)
    sd_ = jax.nn.sigmoid(jnp.dot(hd, dw2_ref[...], precision=_HI) + db2_ref[...])
    ad_ref[...] = sd_ / (jnp.sum(sd_) + 1e-8)


def _mlps(shared, feature_drug, attW1, attb1, attW2, attb2,
          cW1, cb1, cW2, cb2, Wd, bd, dW1, db1, dW2, db2):
    n = shared.shape[0]
    nd = feature_drug.shape[0]
    return pl.pallas_call(
        _mlp_body,
        out_shape=[jax.ShapeDtypeStruct((1, 3), jnp.float32),
                   jax.ShapeDtypeStruct((n, 1), jnp.float32),
                   jax.ShapeDtypeStruct((nd, 1), jnp.float32)],
    )(shared, feature_drug, attW1, attb1.reshape(1, -1), attW2,
      attb2.reshape(1, -1), cW1, cb1.reshape(1, -1), cW2, cb2.reshape(1, -1),
      Wd, bd.reshape(1, -1), dW1, db1.reshape(1, -1), dW2, db2.reshape(1, -1))


def _topk_filter(weighted, vals_src, k):
    filt = jnp.zeros_like(weighted)
    wk = weighted
    for _ in range(k):
        mx = jnp.max(wk, axis=1, keepdims=True)
        sel = wk >= mx
        filt = filt + jnp.where(sel, vals_src, 0.0)
        wk = jnp.where(sel, -jnp.inf, wk)
    return filt


def _fuse_body(w3_ref, kg_ref, kc_ref, km_ref, att_ref, kcw_ref, kcb_ref,
               out_ref):
    fused = (w3_ref[0] * kg_ref[...] + w3_ref[1] * kc_ref[...]
             + w3_ref[2] * km_ref[...])
    weighted = fused * att_ref[...]
    filt = _topk_filter(weighted, fused, _TOPK)
    nt = (((1,), (0,)), ((), ()))
    out_ref[...] = _dot3(filt, kcw_ref[...], nt) + kcb_ref[...]


def _fuse_topk(w3, kg, kc, km, att_row, kcW, kcb, block=128):
    n = kg.shape[0]
    sd = kcW.shape[1]
    grid = (n // block,)
    ksp = pl.BlockSpec((block, n), lambda i: (i, 0))
    return pl.pallas_call(
        _fuse_body,
        grid=grid,
        in_specs=[pl.BlockSpec(memory_space=pltpu.SMEM),
                  ksp, ksp, ksp,
                  pl.BlockSpec((1, n), lambda i: (0, 0)),
                  pl.BlockSpec((n, sd), lambda i: (0, 0)),
                  pl.BlockSpec((1, sd), lambda i: (0, 0))],
        out_specs=pl.BlockSpec((block, sd), lambda i: (i, 0)),
        out_shape=jax.ShapeDtypeStruct((n, sd), jnp.float32),
    )(w3, kg, kc, km, att_row, kcW, kcb.reshape(1, -1))


def _drug_body(fd_ref, att_ref, kdw_ref, kdb_ref, out_ref):
    fd = fd_ref[...]
    dims = (((1,), (1,)), ((), ()))
    inter = _dot1(fd, fd, dims)
    s = jnp.sum(fd, axis=1, keepdims=True)
    jac = inter / (s + s.T - inter + 1e-8)
    weighted = jac * att_ref[...]
    filt = _topk_filter(weighted, jac, _TOPK)
    d = jnp.sum(filt, axis=1, keepdims=True)
    dinv = jax.lax.rsqrt(d + 1e-8)
    fn = filt * dinv * dinv.T
    nt = (((1,), (0,)), ((), ()))
    out_ref[...] = _dot3(fn, kdw_ref[...], nt) + kdb_ref[...]


def _drug_branch(feature_drug, att_row, kdW, kdb):
    nd = feature_drug.shape[0]
    sd = kdW.shape[1]
    return pl.pallas_call(
        _drug_body,
        out_shape=jax.ShapeDtypeStruct((nd, sd), jnp.float32),
    )(feature_drug, att_row, kdW, kdb.reshape(1, -1))


@jax.jit
def kernel(gene, cna, mutation, feature_drug, Wg, bg, Wc, bc, Wm, bm, Wd, bd,
           weights3, attW1, attb1, attW2, attb2, cW1, cb1, cW2, cb2,
           dW1, db1, dW2, db2, kcW, kcb, kdW, kdb):
    gene_n, cna_n = _z_normalize(gene, cna)
    kg, kc, km, shared, norm_sq = _cell_kernels(
        gene_n, cna_n, mutation, Wg, bg, Wc, bc, Wm, bm)
    dyn_sum, att_cell, att_drug = _mlps(
        shared, feature_drug, attW1, attb1, attW2, attb2,
        cW1, cb1, cW2, cb2, Wd, bd, dW1, db1, dW2, db2)

    n_cell = gene.shape[0]
    stat = jax.nn.softmax(weights3)
    dyn_mean = dyn_sum[0] / n_cell
    w = 0.5 * stat + 0.5 * dyn_mean
    w3 = w / (jnp.sqrt(norm_sq) + 1e-8)

    cell_feat = _fuse_topk(w3, kg, kc, km,
                           att_cell.reshape(1, -1), kcW, kcb)
    drug_feat = _drug_branch(feature_drug, att_drug.reshape(1, -1), kdW, kdb)
    return jnp.concatenate([cell_feat, drug_feat], axis=0)


# speedup vs baseline: 8.7877x; 1.2311x over previous
import functools

import jax
import jax.numpy as jnp
from jax.experimental import pallas as pl
from jax.experimental.pallas import tpu as pltpu

_SIGMA = 23.0
_TOPK = 10
_HI = jax.lax.Precision.HIGHEST
_NT = (((1,), (1,)), ((), ()))
_NN = (((1,), (0,)), ((), ()))


def _split(a):
    hi = a.astype(jnp.bfloat16)
    lo = (a - hi.astype(jnp.float32)).astype(jnp.bfloat16)
    return hi, lo


def _dot3(a, b, dims):
    ah, al = _split(a)
    bh, bl = _split(b)
    return _dot3p(ah, al, bh, bl, dims)


def _dot3p(ah, al, bh, bl, dims):
    f = functools.partial(jax.lax.dot_general, dimension_numbers=dims,
                          preferred_element_type=jnp.float32)
    return f(ah, bh) + (f(ah, bl) + f(al, bh))


def _dot1(a, b, dims):
    return jax.lax.dot_general(a.astype(jnp.bfloat16), b.astype(jnp.bfloat16),
                               dimension_numbers=dims,
                               preferred_element_type=jnp.float32)


def _stats_body(g_ref, c_ref, gmu_ref, gsd_ref, cmu_ref, csd_ref):
    for src_, mu_ref, sd_ref in ((g_ref, gmu_ref, gsd_ref),
                                 (c_ref, cmu_ref, csd_ref)):
        x = src_[...]
        n = x.shape[0]
        mu = jnp.sum(x, axis=0, keepdims=True) * (1.0 / n)
        s2 = jnp.sum(x * x, axis=0, keepdims=True) * (1.0 / n)
        mu_ref[...] = mu
        sd_ref[...] = jnp.sqrt(jnp.maximum(s2 - mu * mu, 0.0))


def _col_stats(gene, cna):
    f = gene.shape[1]
    v = jax.ShapeDtypeStruct((1, f), jnp.float32)
    return pl.pallas_call(_stats_body, out_shape=[v, v, v, v])(gene, cna)


def _prep_body(g_ref, c_ref, m_ref, gmu_ref, gsd_ref, cmu_ref, csd_ref,
               wg_ref, bg_ref, wc_ref, bc_ref, wm_ref, bm_ref,
               gh_ref, gl_ref, ch_ref, cl_ref, mh_ref,
               sq_ref, ms_ref, sh_ref):
    gn = (g_ref[...] - gmu_ref[...]) / (gsd_ref[...] + 1e-8)
    gh, gl = _split(gn)
    gh_ref[...] = gh
    gl_ref[...] = gl
    sq_ref[...] = jnp.sum(gn * gn, axis=1, keepdims=True)

    cn = (c_ref[...] - cmu_ref[...]) / (csd_ref[...] + 1e-8)
    ch, cl = _split(cn)
    ch_ref[...] = ch
    cl_ref[...] = cl

    m = m_ref[...]
    mh_ref[...] = m.astype(jnp.bfloat16)
    ms_ref[...] = jnp.sum(m, axis=1, keepdims=True)

    ge = _dot3(gn, wg_ref[...], _NN) + bg_ref[...]
    ce = _dot3(cn, wc_ref[...], _NN) + bc_ref[...]
    me = _dot1(m, wm_ref[...], _NN) + bm_ref[...]
    sh_ref[...] = (ge + ce + me) * (1.0 / 3.0)


def _prep(gene, cna, mutation, Wg, bg, Wc, bc, Wm, bm, block=1024):
    n, f = gene.shape
    sd = Wg.shape[1]
    bf = jnp.bfloat16
    gmu, gsd, cmu, csd = _col_stats(gene, cna)
    grid = (n // block,)
    row = pl.BlockSpec((block, f), lambda i: (i, 0))
    vec = pl.BlockSpec((1, f), lambda i: (0, 0))
    wsp = pl.BlockSpec((f, sd), lambda i: (0, 0))
    bsp = pl.BlockSpec((1, sd), lambda i: (0, 0))
    return pl.pallas_call(
        _prep_body,
        grid=grid,
        in_specs=[row, row, row, vec, vec, vec, vec,
                  wsp, bsp, wsp, bsp, wsp, bsp],
        out_specs=[row, row, row, row, row,
                   pl.BlockSpec((block, 1), lambda i: (i, 0)),
                   pl.BlockSpec((block, 1), lambda i: (i, 0)),
                   pl.BlockSpec((block, sd), lambda i: (i, 0))],
        out_shape=[jax.ShapeDtypeStruct((n, f), bf),
                   jax.ShapeDtypeStruct((n, f), bf),
                   jax.ShapeDtypeStruct((n, f), bf),
                   jax.ShapeDtypeStruct((n, f), bf),
                   jax.ShapeDtypeStruct((n, f), bf),
                   jax.ShapeDtypeStruct((n, 1), jnp.float32),
                   jax.ShapeDtypeStruct((n, 1), jnp.float32),
                   jax.ShapeDtypeStruct((n, sd), jnp.float32)],
    )(gene, cna, mutation, gmu, gsd, cmu, csd,
      Wg, bg.reshape(1, -1), Wc, bc.reshape(1, -1), Wm, bm.reshape(1, -1))


def _kernels_body(ghr_ref, ghc_ref, glr_ref, glc_ref,
                  chr_ref, chc_ref, clr_ref, clc_ref,
                  mhr_ref, mhc_ref, sqr_ref, sqc_ref, msr_ref, msc_ref,
                  kg_ref, kc_ref, km_ref, ng_ref, nc_ref, nm_ref):
    i = pl.program_id(0)
    j = pl.program_id(1)

    @pl.when((i == 0) & (j == 0))
    def _():
        ng_ref[...] = jnp.zeros_like(ng_ref)
        nc_ref[...] = jnp.zeros_like(nc_ref)
        nm_ref[...] = jnp.zeros_like(nm_ref)

    gram_g = _dot3p(ghr_ref[...], glr_ref[...], ghc_ref[...], glc_ref[...],
                    _NT)
    d2 = jnp.maximum(sqr_ref[...] + sqc_ref[...].T - 2.0 * gram_g, 0.0)
    kg = jnp.exp(d2 * (-1.0 / (2.0 * _SIGMA * _SIGMA)))
    kg_ref[...] = kg.astype(jnp.bfloat16)

    p = _dot3p(chr_ref[...], clr_ref[...], chc_ref[...], clc_ref[...],
               _NT) + 1.0
    kc = p * p * p
    kc_ref[...] = kc.astype(jnp.bfloat16)

    inter = jax.lax.dot_general(mhr_ref[...], mhc_ref[...], _NT,
                                preferred_element_type=jnp.float32)
    km = inter / (msr_ref[...] + msc_ref[...].T - inter + 1e-8)
    km_ref[...] = km.astype(jnp.bfloat16)

    ng_ref[...] += jnp.sum(kg * kg, axis=0, keepdims=True)
    nc_ref[...] += jnp.sum(kc * kc, axis=0, keepdims=True)
    nm_ref[...] += jnp.sum(km * km, axis=0, keepdims=True)


def _cell_kernels(gh, gl, ch, cl, mh, sq, ms, tile=512):
    n, f = gh.shape
    grid = (n // tile, n // tile)
    row = pl.BlockSpec((tile, f), lambda i, j: (i, 0))
    col = pl.BlockSpec((tile, f), lambda i, j: (j, 0))
    vrow = pl.BlockSpec((tile, 1), lambda i, j: (i, 0))
    vcol = pl.BlockSpec((tile, 1), lambda i, j: (j, 0))
    out = pl.BlockSpec((tile, tile), lambda i, j: (i, j))
    nsp = pl.BlockSpec((1, tile), lambda i, j: (0, 0))
    bf = jnp.bfloat16
    return pl.pallas_call(
        _kernels_body,
        grid=grid,
        in_specs=[row, col, row, col, row, col, row, col, row, col,
                  vrow, vcol, vrow, vcol],
        out_specs=[out, out, out, nsp, nsp, nsp],
        out_shape=[jax.ShapeDtypeStruct((n, n), bf),
                   jax.ShapeDtypeStruct((n, n), bf),
                   jax.ShapeDtypeStruct((n, n), bf),
                   jax.ShapeDtypeStruct((1, tile), jnp.float32),
                   jax.ShapeDtypeStruct((1, tile), jnp.float32),
                   jax.ShapeDtypeStruct((1, tile), jnp.float32)],
    )(gh, gh, gl, gl, ch, ch, cl, cl, mh, mh, sq, sq, ms, ms)


def _mlp_body(sh_ref, fd_ref, aw1_ref, ab1_ref, aw2_ref, ab2_ref,
              cw1_ref, cb1_ref, cw2_ref, cb2_ref,
              wd_ref, bd_ref, dw1_ref, db1_ref, dw2_ref, db2_ref,
              dyn_ref, ac_ref, ad_ref):
    sh = sh_ref[...]
    h = jnp.maximum(jnp.dot(sh, aw1_ref[...], precision=_HI) + ab1_ref[...], 0.0)
    logits = jnp.dot(h, aw2_ref[...], precision=_HI) + ab2_ref[...]
    mx = jnp.max(logits, axis=1, keepdims=True)
    e = jnp.exp(logits - mx)
    dyn = e / jnp.sum(e, axis=1, keepdims=True)
    dyn_ref[...] = jnp.sum(dyn, axis=0, keepdims=True)

    hc = jnp.maximum(jnp.dot(sh, cw1_ref[...], precision=_HI) + cb1_ref[...], 0.0)
    sc = jax.nn.sigmoid(jnp.dot(hc, cw2_ref[...], precision=_HI) + cb2_ref[...])
    ac_ref[...] = sc / (jnp.sum(sc) + 1e-8)

    de = _dot1(fd_ref[...], wd_ref[...], _NN) + bd_ref[...]
    hd = jnp.maximum(jnp.dot(de, dw1_ref[...], precision=_HI) + db1_ref[...], 0.0)
    sd_ = jax.nn.sigmoid(jnp.dot(hd, dw2_ref[...], precision=_HI) + db2_ref[...])
    ad_ref[...] = sd_ / (jnp.sum(sd_) + 1e-8)


def _mlps(shared, feature_drug, attW1, attb1, attW2, attb2,
          cW1, cb1, cW2, cb2, Wd, bd, dW1, db1, dW2, db2):
    n = shared.shape[0]
    nd = feature_drug.shape[0]
    return pl.pallas_call(
        _mlp_body,
        out_shape=[jax.ShapeDtypeStruct((1, 3), jnp.float32),
                   jax.ShapeDtypeStruct((n, 1), jnp.float32),
                   jax.ShapeDtypeStruct((nd, 1), jnp.float32)],
    )(shared, feature_drug, attW1, attb1.reshape(1, -1), attW2,
      attb2.reshape(1, -1), cW1, cb1.reshape(1, -1), cW2, cb2.reshape(1, -1),
      Wd, bd.reshape(1, -1), dW1, db1.reshape(1, -1), dW2, db2.reshape(1, -1))


def _topk_threshold(weighted, k):
    neg = jnp.float32(-jnp.inf)
    mx = jnp.max(weighted, axis=1, keepdims=True)
    for _ in range(k - 1):
        mx = jnp.max(jnp.where(weighted < mx, weighted, neg),
                     axis=1, keepdims=True)
    return mx


def _fuse_body(w3_ref, kg_ref, kc_ref, km_ref, att_ref, kcw_ref, kcb_ref,
               out_ref):
    fused = (w3_ref[0] * kg_ref[...].astype(jnp.float32)
             + w3_ref[1] * kc_ref[...].astype(jnp.float32)
             + w3_ref[2] * km_ref[...].astype(jnp.float32))
    weighted = fused * att_ref[...]
    thr = _topk_threshold(weighted, _TOPK)
    filt = jnp.where(weighted >= thr, fused, 0.0)
    out_ref[...] = _dot3(filt, kcw_ref[...], _NN) + kcb_ref[...]


def _fuse_topk(w3, kg, kc, km, att_row, kcW, kcb, block=256):
    n = kg.shape[0]
    sd = kcW.shape[1]
    grid = (n // block,)
    ksp = pl.BlockSpec((block, n), lambda i: (i, 0))
    return pl.pallas_call(
        _fuse_body,
        grid=grid,
        in_specs=[pl.BlockSpec(memory_space=pltpu.SMEM),
                  ksp, ksp, ksp,
                  pl.BlockSpec((1, n), lambda i: (0, 0)),
                  pl.BlockSpec((n, sd), lambda i: (0, 0)),
                  pl.BlockSpec((1, sd), lambda i: (0, 0))],
        out_specs=pl.BlockSpec((block, sd), lambda i: (i, 0)),
        out_shape=jax.ShapeDtypeStruct((n, sd), jnp.float32),
    )(w3, kg, kc, km, att_row, kcW, kcb.reshape(1, -1))


def _drug_body(fd_ref, att_ref, kdw_ref, kdb_ref, out_ref):
    fd = fd_ref[...]
    inter = _dot1(fd, fd, _NT)
    s = jnp.sum(fd, axis=1, keepdims=True)
    jac = inter / (s + s.T - inter + 1e-8)
    weighted = jac * att_ref[...]
    thr = _topk_threshold(weighted, _TOPK)
    filt = jnp.where(weighted >= thr, jac, 0.0)
    d = jnp.sum(filt, axis=1, keepdims=True)
    dinv = jax.lax.rsqrt(d + 1e-8)
    fn = filt * dinv * dinv.T
    out_ref[...] = _dot3(fn, kdw_ref[...], _NN) + kdb_ref[...]


def _drug_branch(feature_drug, att_row, kdW, kdb):
    nd = feature_drug.shape[0]
    sd = kdW.shape[1]
    return pl.pallas_call(
        _drug_body,
        out_shape=jax.ShapeDtypeStruct((nd, sd), jnp.float32),
    )(feature_drug, att_row, kdW, kdb.reshape(1, -1))


@jax.jit
def kernel(gene, cna, mutation, feature_drug, Wg, bg, Wc, bc, Wm, bm, Wd, bd,
           weights3, attW1, attb1, attW2, attb2, cW1, cb1, cW2, cb2,
           dW1, db1, dW2, db2, kcW, kcb, kdW, kdb):
    gh, gl, ch, cl, mh, sq, ms, shared = _prep(
        gene, cna, mutation, Wg, bg, Wc, bc, Wm, bm)
    kg, kc, km, ng, nc, nm = _cell_kernels(gh, gl, ch, cl, mh, sq, ms)
    dyn_sum, att_cell, att_drug = _mlps(
        shared, feature_drug, attW1, attb1, attW2, attb2,
        cW1, cb1, cW2, cb2, Wd, bd, dW1, db1, dW2, db2)

    n_cell = gene.shape[0]
    stat = jax.nn.softmax(weights3)
    dyn_mean = dyn_sum[0] / n_cell
    w = 0.5 * stat + 0.5 * dyn_mean
    norms = jnp.sqrt(jnp.stack([jnp.sum(ng), jnp.sum(nc), jnp.sum(nm)]))
    w3 = w / (norms + 1e-8)

    cell_feat = _fuse_topk(w3, kg, kc, km,
                           att_cell.reshape(1, -1), kcW, kcb)
    drug_feat = _drug_branch(feature_drug, att_drug.reshape(1, -1), kdW, kdb)
    return jnp.concatenate([cell_feat, drug_feat], axis=0)


# speedup vs baseline: 10.3544x; 1.1783x over previous
import functools

import jax
import jax.numpy as jnp
from jax.experimental import pallas as pl
from jax.experimental.pallas import tpu as pltpu

_SIGMA = 23.0
_TOPK = 10
_HI = jax.lax.Precision.HIGHEST
_NT = (((1,), (1,)), ((), ()))
_NN = (((1,), (0,)), ((), ()))


def _split(a):
    hi = a.astype(jnp.bfloat16)
    lo = (a - hi.astype(jnp.float32)).astype(jnp.bfloat16)
    return hi, lo


def _dot3(a, b, dims):
    ah, al = _split(a)
    bh, bl = _split(b)
    return _dot3p(ah, al, bh, bl, dims)


def _dot3p(ah, al, bh, bl, dims):
    f = functools.partial(jax.lax.dot_general, dimension_numbers=dims,
                          preferred_element_type=jnp.float32)
    return f(ah, bh) + (f(ah, bl) + f(al, bh))


def _dot1(a, b, dims):
    return jax.lax.dot_general(a.astype(jnp.bfloat16), b.astype(jnp.bfloat16),
                               dimension_numbers=dims,
                               preferred_element_type=jnp.float32)


def _stats_body(g_ref, c_ref, gmu_ref, gsd_ref, cmu_ref, csd_ref):
    for src_, mu_ref, sd_ref in ((g_ref, gmu_ref, gsd_ref),
                                 (c_ref, cmu_ref, csd_ref)):
        x = src_[...]
        n = x.shape[0]
        mu = jnp.sum(x, axis=0, keepdims=True) * (1.0 / n)
        s2 = jnp.sum(x * x, axis=0, keepdims=True) * (1.0 / n)
        mu_ref[...] = mu
        sd_ref[...] = jnp.sqrt(jnp.maximum(s2 - mu * mu, 0.0))


def _col_stats(gene, cna):
    f = gene.shape[1]
    v = jax.ShapeDtypeStruct((1, f), jnp.float32)
    return pl.pallas_call(_stats_body, out_shape=[v, v, v, v])(gene, cna)


def _prep_body(g_ref, c_ref, m_ref, gmu_ref, gsd_ref, cmu_ref, csd_ref,
               wg_ref, bg_ref, wc_ref, bc_ref, wm_ref, bm_ref,
               gh_ref, ch_ref, mh_ref,
               sq_ref, ms_ref, sh_ref):
    gn = (g_ref[...] - gmu_ref[...]) / (gsd_ref[...] + 1e-8)
    gh_ref[...] = gn.astype(jnp.bfloat16)
    sq_ref[...] = jnp.sum(gn * gn, axis=1, keepdims=True)

    cn = (c_ref[...] - cmu_ref[...]) / (csd_ref[...] + 1e-8)
    ch_ref[...] = cn.astype(jnp.bfloat16)

    m = m_ref[...]
    mh_ref[...] = m.astype(jnp.bfloat16)
    ms_ref[...] = jnp.sum(m, axis=1, keepdims=True)

    ge = _dot3(gn, wg_ref[...], _NN) + bg_ref[...]
    ce = _dot3(cn, wc_ref[...], _NN) + bc_ref[...]
    me = _dot1(m, wm_ref[...], _NN) + bm_ref[...]
    sh_ref[...] = (ge + ce + me) * (1.0 / 3.0)


def _prep(gene, cna, mutation, Wg, bg, Wc, bc, Wm, bm, block=1024):
    n, f = gene.shape
    sd = Wg.shape[1]
    bf = jnp.bfloat16
    gmu, gsd, cmu, csd = _col_stats(gene, cna)
    grid = (n // block,)
    row = pl.BlockSpec((block, f), lambda i: (i, 0))
    vec = pl.BlockSpec((1, f), lambda i: (0, 0))
    wsp = pl.BlockSpec((f, sd), lambda i: (0, 0))
    bsp = pl.BlockSpec((1, sd), lambda i: (0, 0))
    return pl.pallas_call(
        _prep_body,
        grid=grid,
        in_specs=[row, row, row, vec, vec, vec, vec,
                  wsp, bsp, wsp, bsp, wsp, bsp],
        out_specs=[row, row, row,
                   pl.BlockSpec((block, 1), lambda i: (i, 0)),
                   pl.BlockSpec((block, 1), lambda i: (i, 0)),
                   pl.BlockSpec((block, sd), lambda i: (i, 0))],
        out_shape=[jax.ShapeDtypeStruct((n, f), bf),
                   jax.ShapeDtypeStruct((n, f), bf),
                   jax.ShapeDtypeStruct((n, f), bf),
                   jax.ShapeDtypeStruct((n, 1), jnp.float32),
                   jax.ShapeDtypeStruct((n, 1), jnp.float32),
                   jax.ShapeDtypeStruct((n, sd), jnp.float32)],
    )(gene, cna, mutation, gmu, gsd, cmu, csd,
      Wg, bg.reshape(1, -1), Wc, bc.reshape(1, -1), Wm, bm.reshape(1, -1))


def _kernels_body(ghr_ref, ghc_ref, chr_ref, chc_ref,
                  mhr_ref, mhc_ref, sqr_ref, sqc_ref, msr_ref, msc_ref,
                  kg_ref, kc_ref, km_ref, ng_ref, nc_ref, nm_ref):
    i = pl.program_id(0)
    j = pl.program_id(1)

    @pl.when((i == 0) & (j == 0))
    def _():
        ng_ref[...] = jnp.zeros_like(ng_ref)
        nc_ref[...] = jnp.zeros_like(nc_ref)
        nm_ref[...] = jnp.zeros_like(nm_ref)

    gram_g = jax.lax.dot_general(ghr_ref[...], ghc_ref[...], _NT,
                                 preferred_element_type=jnp.float32)
    d2 = jnp.maximum(sqr_ref[...] + sqc_ref[...].T - 2.0 * gram_g, 0.0)
    kg = jnp.exp(d2 * (-1.0 / (2.0 * _SIGMA * _SIGMA)))
    kg_ref[...] = kg.astype(jnp.bfloat16)

    p = jax.lax.dot_general(chr_ref[...], chc_ref[...], _NT,
                            preferred_element_type=jnp.float32) + 1.0
    kc = p * p * p
    kc_ref[...] = kc.astype(jnp.bfloat16)

    inter = jax.lax.dot_general(mhr_ref[...], mhc_ref[...], _NT,
                                preferred_element_type=jnp.float32)
    km = inter / (msr_ref[...] + msc_ref[...].T - inter + 1e-8)
    km_ref[...] = km.astype(jnp.bfloat16)

    ng_ref[...] += jnp.sum(kg * kg, axis=0, keepdims=True)
    nc_ref[...] += jnp.sum(kc * kc, axis=0, keepdims=True)
    nm_ref[...] += jnp.sum(km * km, axis=0, keepdims=True)


def _cell_kernels(gh, ch, mh, sq, ms, tile=512):
    n, f = gh.shape
    grid = (n // tile, n // tile)
    row = pl.BlockSpec((tile, f), lambda i, j: (i, 0))
    col = pl.BlockSpec((tile, f), lambda i, j: (j, 0))
    vrow = pl.BlockSpec((tile, 1), lambda i, j: (i, 0))
    vcol = pl.BlockSpec((tile, 1), lambda i, j: (j, 0))
    out = pl.BlockSpec((tile, tile), lambda i, j: (i, j))
    nsp = pl.BlockSpec((1, tile), lambda i, j: (0, 0))
    bf = jnp.bfloat16
    return pl.pallas_call(
        _kernels_body,
        grid=grid,
        in_specs=[row, col, row, col, row, col,
                  vrow, vcol, vrow, vcol],
        out_specs=[out, out, out, nsp, nsp, nsp],
        out_shape=[jax.ShapeDtypeStruct((n, n), bf),
                   jax.ShapeDtypeStruct((n, n), bf),
                   jax.ShapeDtypeStruct((n, n), bf),
                   jax.ShapeDtypeStruct((1, tile), jnp.float32),
                   jax.ShapeDtypeStruct((1, tile), jnp.float32),
                   jax.ShapeDtypeStruct((1, tile), jnp.float32)],
    )(gh, gh, ch, ch, mh, mh, sq, sq, ms, ms)


def _mlp_body(sh_ref, fd_ref, aw1_ref, ab1_ref, aw2_ref, ab2_ref,
              cw1_ref, cb1_ref, cw2_ref, cb2_ref,
              wd_ref, bd_ref, dw1_ref, db1_ref, dw2_ref, db2_ref,
              dyn_ref, ac_ref, ad_ref):
    sh = sh_ref[...]
    h = jnp.maximum(jnp.dot(sh, aw1_ref[...], precision=_HI) + ab1_ref[...], 0.0)
    logits = jnp.dot(h, aw2_ref[...], precision=_HI) + ab2_ref[...]
    mx = jnp.max(logits, axis=1, keepdims=True)
    e = jnp.exp(logits - mx)
    dyn = e / jnp.sum(e, axis=1, keepdims=True)
    dyn_ref[...] = jnp.sum(dyn, axis=0, keepdims=True)

    hc = jnp.maximum(jnp.dot(sh, cw1_ref[...], precision=_HI) + cb1_ref[...], 0.0)
    sc = jax.nn.sigmoid(jnp.dot(hc, cw2_ref[...], precision=_HI) + cb2_ref[...])
    ac_ref[...] = sc / (jnp.sum(sc) + 1e-8)

    de = _dot1(fd_ref[...], wd_ref[...], _NN) + bd_ref[...]
    hd = jnp.maximum(jnp.dot(de, dw1_ref[...], precision=_HI) + db1_ref[...], 0.0)
    sd_ = jax.nn.sigmoid(jnp.dot(hd, dw2_ref[...], precision=_HI) + db2_ref[...])
    ad_ref[...] = sd_ / (jnp.sum(sd_) + 1e-8)


def _mlps(shared, feature_drug, attW1, attb1, attW2, attb2,
          cW1, cb1, cW2, cb2, Wd, bd, dW1, db1, dW2, db2):
    n = shared.shape[0]
    nd = feature_drug.shape[0]
    return pl.pallas_call(
        _mlp_body,
        out_shape=[jax.ShapeDtypeStruct((1, 3), jnp.float32),
                   jax.ShapeDtypeStruct((n, 1), jnp.float32),
                   jax.ShapeDtypeStruct((nd, 1), jnp.float32)],
    )(shared, feature_drug, attW1, attb1.reshape(1, -1), attW2,
      attb2.reshape(1, -1), cW1, cb1.reshape(1, -1), cW2, cb2.reshape(1, -1),
      Wd, bd.reshape(1, -1), dW1, db1.reshape(1, -1), dW2, db2.reshape(1, -1))


def _topk_threshold(weighted, k):
    neg = jnp.float32(-jnp.inf)
    mx = jnp.max(weighted, axis=1, keepdims=True)
    for _ in range(k - 1):
        mx = jnp.max(jnp.where(weighted < mx, weighted, neg),
                     axis=1, keepdims=True)
    return mx


def _fuse_body(w3_ref, kg_ref, kc_ref, km_ref, att_ref, kcw_ref, kcb_ref,
               out_ref):
    fused = (w3_ref[0] * kg_ref[...].astype(jnp.float32)
             + w3_ref[1] * kc_ref[...].astype(jnp.float32)
             + w3_ref[2] * km_ref[...].astype(jnp.float32))
    weighted = fused * att_ref[...]
    thr = _topk_threshold(weighted, _TOPK)
    filt = jnp.where(weighted >= thr, fused, 0.0)
    out_ref[...] = _dot3(filt, kcw_ref[...], _NN) + kcb_ref[...]


def _fuse_topk(w3, kg, kc, km, att_row, kcW, kcb, block=256):
    n = kg.shape[0]
    sd = kcW.shape[1]
    grid = (n // block,)
    ksp = pl.BlockSpec((block, n), lambda i: (i, 0))
    return pl.pallas_call(
        _fuse_body,
        grid=grid,
        in_specs=[pl.BlockSpec(memory_space=pltpu.SMEM),
                  ksp, ksp, ksp,
                  pl.BlockSpec((1, n), lambda i: (0, 0)),
                  pl.BlockSpec((n, sd), lambda i: (0, 0)),
                  pl.BlockSpec((1, sd), lambda i: (0, 0))],
        out_specs=pl.BlockSpec((block, sd), lambda i: (i, 0)),
        out_shape=jax.ShapeDtypeStruct((n, sd), jnp.float32),
    )(w3, kg, kc, km, att_row, kcW, kcb.reshape(1, -1))


def _drug_body(fd_ref, att_ref, kdw_ref, kdb_ref, out_ref):
    fd = fd_ref[...]
    inter = _dot1(fd, fd, _NT)
    s = jnp.sum(fd, axis=1, keepdims=True)
    jac = inter / (s + s.T - inter + 1e-8)
    weighted = jac * att_ref[...]
    thr = _topk_threshold(weighted, _TOPK)
    filt = jnp.where(weighted >= thr, jac, 0.0)
    d = jnp.sum(filt, axis=1, keepdims=True)
    dinv = jax.lax.rsqrt(d + 1e-8)
    fn = filt * dinv * dinv.T
    out_ref[...] = _dot3(fn, kdw_ref[...], _NN) + kdb_ref[...]


def _drug_branch(feature_drug, att_row, kdW, kdb):
    nd = feature_drug.shape[0]
    sd = kdW.shape[1]
    return pl.pallas_call(
        _drug_body,
        out_shape=jax.ShapeDtypeStruct((nd, sd), jnp.float32),
    )(feature_drug, att_row, kdW, kdb.reshape(1, -1))


@jax.jit
def kernel(gene, cna, mutation, feature_drug, Wg, bg, Wc, bc, Wm, bm, Wd, bd,
           weights3, attW1, attb1, attW2, attb2, cW1, cb1, cW2, cb2,
           dW1, db1, dW2, db2, kcW, kcb, kdW, kdb):
    gh, ch, mh, sq, ms, shared = _prep(
        gene, cna, mutation, Wg, bg, Wc, bc, Wm, bm)
    kg, kc, km, ng, nc, nm = _cell_kernels(gh, ch, mh, sq, ms)
    dyn_sum, att_cell, att_drug = _mlps(
        shared, feature_drug, attW1, attb1, attW2, attb2,
        cW1, cb1, cW2, cb2, Wd, bd, dW1, db1, dW2, db2)

    n_cell = gene.shape[0]
    stat = jax.nn.softmax(weights3)
    dyn_mean = dyn_sum[0] / n_cell
    w = 0.5 * stat + 0.5 * dyn_mean
    norms = jnp.sqrt(jnp.stack([jnp.sum(ng), jnp.sum(nc), jnp.sum(nm)]))
    w3 = w / (norms + 1e-8)

    cell_feat = _fuse_topk(w3, kg, kc, km,
                           att_cell.reshape(1, -1), kcW, kcb)
    drug_feat = _drug_branch(feature_drug, att_drug.reshape(1, -1), kdW, kdb)
    return jnp.concatenate([cell_feat, drug_feat], axis=0)


# speedup vs baseline: 11.7931x; 1.1389x over previous
import functools

import jax
import jax.numpy as jnp
from jax.experimental import pallas as pl
from jax.experimental.pallas import tpu as pltpu

_SIGMA = 23.0
_TOPK = 10
_HI = jax.lax.Precision.HIGHEST
_NT = (((1,), (1,)), ((), ()))
_NN = (((1,), (0,)), ((), ()))


def _split(a):
    hi = a.astype(jnp.bfloat16)
    lo = (a - hi.astype(jnp.float32)).astype(jnp.bfloat16)
    return hi, lo


def _dot3(a, b, dims):
    ah, al = _split(a)
    bh, bl = _split(b)
    return _dot3p(ah, al, bh, bl, dims)


def _dot3p(ah, al, bh, bl, dims):
    f = functools.partial(jax.lax.dot_general, dimension_numbers=dims,
                          preferred_element_type=jnp.float32)
    return f(ah, bh) + (f(ah, bl) + f(al, bh))


def _dot1(a, b, dims):
    return jax.lax.dot_general(a.astype(jnp.bfloat16), b.astype(jnp.bfloat16),
                               dimension_numbers=dims,
                               preferred_element_type=jnp.float32)


def _stats_body(g_ref, c_ref, gmu_ref, gsd_ref, cmu_ref, csd_ref):
    for src_, mu_ref, sd_ref in ((g_ref, gmu_ref, gsd_ref),
                                 (c_ref, cmu_ref, csd_ref)):
        x = src_[...]
        n = x.shape[0]
        mu = jnp.sum(x, axis=0, keepdims=True) * (1.0 / n)
        s2 = jnp.sum(x * x, axis=0, keepdims=True) * (1.0 / n)
        mu_ref[...] = mu
        sd_ref[...] = jnp.sqrt(jnp.maximum(s2 - mu * mu, 0.0))


def _col_stats(gene, cna):
    f = gene.shape[1]
    v = jax.ShapeDtypeStruct((1, f), jnp.float32)
    return pl.pallas_call(_stats_body, out_shape=[v, v, v, v])(gene, cna)


def _prep_body(g_ref, c_ref, m_ref, gmu_ref, gsd_ref, cmu_ref, csd_ref,
               wg_ref, bg_ref, wc_ref, bc_ref, wm_ref, bm_ref,
               gh_ref, ch_ref, mh_ref,
               sq_ref, ms_ref, sh_ref):
    gn = (g_ref[...] - gmu_ref[...]) / (gsd_ref[...] + 1e-8)
    gh_ref[...] = gn.astype(jnp.bfloat16)
    sq_ref[...] = jnp.sum(gn * gn, axis=1, keepdims=True)

    cn = (c_ref[...] - cmu_ref[...]) / (csd_ref[...] + 1e-8)
    ch_ref[...] = cn.astype(jnp.bfloat16)

    m = m_ref[...]
    mh_ref[...] = m.astype(jnp.bfloat16)
    ms_ref[...] = jnp.sum(m, axis=1, keepdims=True)

    ge = _dot3(gn, wg_ref[...], _NN) + bg_ref[...]
    ce = _dot3(cn, wc_ref[...], _NN) + bc_ref[...]
    me = _dot1(m, wm_ref[...], _NN) + bm_ref[...]
    sh_ref[...] = (ge + ce + me) * (1.0 / 3.0)


def _prep(gene, cna, mutation, Wg, bg, Wc, bc, Wm, bm, block=1024):
    n, f = gene.shape
    sd = Wg.shape[1]
    bf = jnp.bfloat16
    gmu, gsd, cmu, csd = _col_stats(gene, cna)
    grid = (n // block,)
    row = pl.BlockSpec((block, f), lambda i: (i, 0))
    vec = pl.BlockSpec((1, f), lambda i: (0, 0))
    wsp = pl.BlockSpec((f, sd), lambda i: (0, 0))
    bsp = pl.BlockSpec((1, sd), lambda i: (0, 0))
    return pl.pallas_call(
        _prep_body,
        grid=grid,
        in_specs=[row, row, row, vec, vec, vec, vec,
                  wsp, bsp, wsp, bsp, wsp, bsp],
        out_specs=[row, row, row,
                   pl.BlockSpec((block, 1), lambda i: (i, 0)),
                   pl.BlockSpec((block, 1), lambda i: (i, 0)),
                   pl.BlockSpec((block, sd), lambda i: (i, 0))],
        out_shape=[jax.ShapeDtypeStruct((n, f), bf),
                   jax.ShapeDtypeStruct((n, f), bf),
                   jax.ShapeDtypeStruct((n, f), bf),
                   jax.ShapeDtypeStruct((n, 1), jnp.float32),
                   jax.ShapeDtypeStruct((n, 1), jnp.float32),
                   jax.ShapeDtypeStruct((n, sd), jnp.float32)],
    )(gene, cna, mutation, gmu, gsd, cmu, csd,
      Wg, bg.reshape(1, -1), Wc, bc.reshape(1, -1), Wm, bm.reshape(1, -1))


def _kernels_body(ghr_ref, ghc_ref, chr_ref, chc_ref,
                  mhr_ref, mhc_ref, sqr_ref, sqc_ref, msr_ref, msc_ref,
                  kg_ref, kc_ref, km_ref, ng_ref, nc_ref, nm_ref):
    i = pl.program_id(0)
    j = pl.program_id(1)

    @pl.when((i == 0) & (j == 0))
    def _():
        ng_ref[...] = jnp.zeros_like(ng_ref)
        nc_ref[...] = jnp.zeros_like(nc_ref)
        nm_ref[...] = jnp.zeros_like(nm_ref)

    gram_g = jax.lax.dot_general(ghr_ref[...], ghc_ref[...], _NT,
                                 preferred_element_type=jnp.float32)
    d2 = jnp.maximum(sqr_ref[...] + sqc_ref[...].T - 2.0 * gram_g, 0.0)
    kg = jnp.exp(d2 * (-1.0 / (2.0 * _SIGMA * _SIGMA)))
    kg_ref[...] = kg.astype(jnp.bfloat16)

    p = jax.lax.dot_general(chr_ref[...], chc_ref[...], _NT,
                            preferred_element_type=jnp.float32) + 1.0
    kc = p * p * p
    kc_ref[...] = kc.astype(jnp.bfloat16)

    inter = jax.lax.dot_general(mhr_ref[...], mhc_ref[...], _NT,
                                preferred_element_type=jnp.float32)
    km = inter / (msr_ref[...] + msc_ref[...].T - inter + 1e-8)
    km_ref[...] = km.astype(jnp.bfloat16)

    ng_ref[...] += jnp.sum(kg * kg, axis=0, keepdims=True)
    nc_ref[...] += jnp.sum(kc * kc, axis=0, keepdims=True)
    nm_ref[...] += jnp.sum(km * km, axis=0, keepdims=True)


def _cell_kernels(gh, ch, mh, sq, ms, tile=1024):
    n, f = gh.shape
    grid = (n // tile, n // tile)
    row = pl.BlockSpec((tile, f), lambda i, j: (i, 0))
    col = pl.BlockSpec((tile, f), lambda i, j: (j, 0))
    vrow = pl.BlockSpec((tile, 1), lambda i, j: (i, 0))
    vcol = pl.BlockSpec((tile, 1), lambda i, j: (j, 0))
    out = pl.BlockSpec((tile, tile), lambda i, j: (i, j))
    nsp = pl.BlockSpec((1, tile), lambda i, j: (0, 0))
    bf = jnp.bfloat16
    return pl.pallas_call(
        _kernels_body,
        grid=grid,
        in_specs=[row, col, row, col, row, col,
                  vrow, vcol, vrow, vcol],
        out_specs=[out, out, out, nsp, nsp, nsp],
        out_shape=[jax.ShapeDtypeStruct((n, n), bf),
                   jax.ShapeDtypeStruct((n, n), bf),
                   jax.ShapeDtypeStruct((n, n), bf),
                   jax.ShapeDtypeStruct((1, tile), jnp.float32),
                   jax.ShapeDtypeStruct((1, tile), jnp.float32),
                   jax.ShapeDtypeStruct((1, tile), jnp.float32)],
    )(gh, gh, ch, ch, mh, mh, sq, sq, ms, ms)


def _mlp_body(sh_ref, fd_ref, aw1_ref, ab1_ref, aw2_ref, ab2_ref,
              cw1_ref, cb1_ref, cw2_ref, cb2_ref,
              wd_ref, bd_ref, dw1_ref, db1_ref, dw2_ref, db2_ref,
              dyn_ref, ac_ref, ad_ref):
    sh = sh_ref[...]
    h = jnp.maximum(jnp.dot(sh, aw1_ref[...], precision=_HI) + ab1_ref[...], 0.0)
    logits = jnp.dot(h, aw2_ref[...], precision=_HI) + ab2_ref[...]
    mx = jnp.max(logits, axis=1, keepdims=True)
    e = jnp.exp(logits - mx)
    dyn = e / jnp.sum(e, axis=1, keepdims=True)
    dyn_ref[...] = jnp.sum(dyn, axis=0, keepdims=True)

    hc = jnp.maximum(jnp.dot(sh, cw1_ref[...], precision=_HI) + cb1_ref[...], 0.0)
    sc = jax.nn.sigmoid(jnp.dot(hc, cw2_ref[...], precision=_HI) + cb2_ref[...])
    ac_ref[...] = sc / (jnp.sum(sc) + 1e-8)

    de = _dot1(fd_ref[...], wd_ref[...], _NN) + bd_ref[...]
    hd = jnp.maximum(jnp.dot(de, dw1_ref[...], precision=_HI) + db1_ref[...], 0.0)
    sd_ = jax.nn.sigmoid(jnp.dot(hd, dw2_ref[...], precision=_HI) + db2_ref[...])
    ad_ref[...] = sd_ / (jnp.sum(sd_) + 1e-8)


def _mlps(shared, feature_drug, attW1, attb1, attW2, attb2,
          cW1, cb1, cW2, cb2, Wd, bd, dW1, db1, dW2, db2):
    n = shared.shape[0]
    nd = feature_drug.shape[0]
    return pl.pallas_call(
        _mlp_body,
        out_shape=[jax.ShapeDtypeStruct((1, 3), jnp.float32),
                   jax.ShapeDtypeStruct((n, 1), jnp.float32),
                   jax.ShapeDtypeStruct((nd, 1), jnp.float32)],
    )(shared, feature_drug, attW1, attb1.reshape(1, -1), attW2,
      attb2.reshape(1, -1), cW1, cb1.reshape(1, -1), cW2, cb2.reshape(1, -1),
      Wd, bd.reshape(1, -1), dW1, db1.reshape(1, -1), dW2, db2.reshape(1, -1))


def _topk_threshold(weighted, k):
    neg = jnp.float32(-jnp.inf)
    mx = jnp.max(weighted, axis=1, keepdims=True)
    for _ in range(k - 1):
        mx = jnp.max(jnp.where(weighted < mx, weighted, neg),
                     axis=1, keepdims=True)
    return mx


def _fuse_body(w3_ref, kg_ref, kc_ref, km_ref, att_ref, kcw_ref, kcb_ref,
               out_ref):
    fused = (w3_ref[0] * kg_ref[...].astype(jnp.float32)
             + w3_ref[1] * kc_ref[...].astype(jnp.float32)
             + w3_ref[2] * km_ref[...].astype(jnp.float32))
    weighted = fused * att_ref[...]
    thr = _topk_threshold(weighted, _TOPK)
    filt = jnp.where(weighted >= thr, fused, 0.0)
    out_ref[...] = _dot3(filt, kcw_ref[...], _NN) + kcb_ref[...]


def _fuse_topk(w3, kg, kc, km, att_row, kcW, kcb, block=256):
    n = kg.shape[0]
    sd = kcW.shape[1]
    grid = (n // block,)
    ksp = pl.BlockSpec((block, n), lambda i: (i, 0))
    return pl.pallas_call(
        _fuse_body,
        grid=grid,
        in_specs=[pl.BlockSpec(memory_space=pltpu.SMEM),
                  ksp, ksp, ksp,
                  pl.BlockSpec((1, n), lambda i: (0, 0)),
                  pl.BlockSpec((n, sd), lambda i: (0, 0)),
                  pl.BlockSpec((1, sd), lambda i: (0, 0))],
        out_specs=pl.BlockSpec((block, sd), lambda i: (i, 0)),
        out_shape=jax.ShapeDtypeStruct((n, sd), jnp.float32),
    )(w3, kg, kc, km, att_row, kcW, kcb.reshape(1, -1))


def _drug_body(fd_ref, att_ref, kdw_ref, kdb_ref, out_ref):
    fd = fd_ref[...]
    inter = _dot1(fd, fd, _NT)
    s = jnp.sum(fd, axis=1, keepdims=True)
    jac = inter / (s + s.T - inter + 1e-8)
    weighted = jac * att_ref[...]
    thr = _topk_threshold(weighted, _TOPK)
    filt = jnp.where(weighted >= thr, jac, 0.0)
    d = jnp.sum(filt, axis=1, keepdims=True)
    dinv = jax.lax.rsqrt(d + 1e-8)
    fn = filt * dinv * dinv.T
    out_ref[...] = _dot3(fn, kdw_ref[...], _NN) + kdb_ref[...]


def _drug_branch(feature_drug, att_row, kdW, kdb):
    nd = feature_drug.shape[0]
    sd = kdW.shape[1]
    return pl.pallas_call(
        _drug_body,
        out_shape=jax.ShapeDtypeStruct((nd, sd), jnp.float32),
    )(feature_drug, att_row, kdW, kdb.reshape(1, -1))


@jax.jit
def kernel(gene, cna, mutation, feature_drug, Wg, bg, Wc, bc, Wm, bm, Wd, bd,
           weights3, attW1, attb1, attW2, attb2, cW1, cb1, cW2, cb2,
           dW1, db1, dW2, db2, kcW, kcb, kdW, kdb):
    gh, ch, mh, sq, ms, shared = _prep(
        gene, cna, mutation, Wg, bg, Wc, bc, Wm, bm)
    kg, kc, km, ng, nc, nm = _cell_kernels(gh, ch, mh, sq, ms)
    dyn_sum, att_cell, att_drug = _mlps(
        shared, feature_drug, attW1, attb1, attW2, attb2,
        cW1, cb1, cW2, cb2, Wd, bd, dW1, db1, dW2, db2)

    n_cell = gene.shape[0]
    stat = jax.nn.softmax(weights3)
    dyn_mean = dyn_sum[0] / n_cell
    w = 0.5 * stat + 0.5 * dyn_mean
    norms = jnp.sqrt(jnp.stack([jnp.sum(ng), jnp.sum(nc), jnp.sum(nm)]))
    w3 = w / (norms + 1e-8)

    cell_feat = _fuse_topk(w3, kg, kc, km,
                           att_cell.reshape(1, -1), kcW, kcb)
    drug_feat = _drug_branch(feature_drug, att_drug.reshape(1, -1), kdW, kdb)
    return jnp.concatenate([cell_feat, drug_feat], axis=0)


# speedup vs baseline: 12.0531x; 1.0220x over previous
import functools

import jax
import jax.numpy as jnp
from jax.experimental import pallas as pl
from jax.experimental.pallas import tpu as pltpu

_SIGMA = 23.0
_TOPK = 10
_HI = jax.lax.Precision.HIGHEST
_NT = (((1,), (1,)), ((), ()))
_NN = (((1,), (0,)), ((), ()))


def _split(a):
    hi = a.astype(jnp.bfloat16)
    lo = (a - hi.astype(jnp.float32)).astype(jnp.bfloat16)
    return hi, lo


def _dot3(a, b, dims):
    ah, al = _split(a)
    bh, bl = _split(b)
    return _dot3p(ah, al, bh, bl, dims)


def _dot3p(ah, al, bh, bl, dims):
    f = functools.partial(jax.lax.dot_general, dimension_numbers=dims,
                          preferred_element_type=jnp.float32)
    return f(ah, bh) + (f(ah, bl) + f(al, bh))


def _dot1(a, b, dims):
    return jax.lax.dot_general(a.astype(jnp.bfloat16), b.astype(jnp.bfloat16),
                               dimension_numbers=dims,
                               preferred_element_type=jnp.float32)


def _stats_body(g_ref, c_ref, gmu_ref, gsd_ref, cmu_ref, csd_ref):
    for src_, mu_ref, sd_ref in ((g_ref, gmu_ref, gsd_ref),
                                 (c_ref, cmu_ref, csd_ref)):
        x = src_[...]
        n = x.shape[0]
        mu = jnp.sum(x, axis=0, keepdims=True) * (1.0 / n)
        s2 = jnp.sum(x * x, axis=0, keepdims=True) * (1.0 / n)
        mu_ref[...] = mu
        sd_ref[...] = jnp.sqrt(jnp.maximum(s2 - mu * mu, 0.0))


def _col_stats(gene, cna):
    f = gene.shape[1]
    v = jax.ShapeDtypeStruct((1, f), jnp.float32)
    return pl.pallas_call(_stats_body, out_shape=[v, v, v, v])(gene, cna)


def _prep_body(g_ref, c_ref, m_ref, gmu_ref, gsd_ref, cmu_ref, csd_ref,
               wg_ref, bg_ref, wc_ref, bc_ref, wm_ref, bm_ref,
               aw1_ref, ab1_ref, aw2_ref, ab2_ref,
               cw1_ref, cb1_ref, cw2_ref, cb2_ref,
               gh_ref, ch_ref, mh_ref,
               sq_ref, ms_ref, dyn_ref, ac_ref):
    gn = (g_ref[...] - gmu_ref[...]) / (gsd_ref[...] + 1e-8)
    gh_ref[...] = gn.astype(jnp.bfloat16)
    sq_ref[...] = jnp.sum(gn * gn, axis=1, keepdims=True)

    cn = (c_ref[...] - cmu_ref[...]) / (csd_ref[...] + 1e-8)
    ch_ref[...] = cn.astype(jnp.bfloat16)

    m = m_ref[...]
    mh_ref[...] = m.astype(jnp.bfloat16)
    ms_ref[...] = jnp.sum(m, axis=1, keepdims=True)

    ge = _dot3(gn, wg_ref[...], _NN) + bg_ref[...]
    ce = _dot3(cn, wc_ref[...], _NN) + bc_ref[...]
    me = _dot1(m, wm_ref[...], _NN) + bm_ref[...]
    sh = (ge + ce + me) * (1.0 / 3.0)

    h = jnp.maximum(jnp.dot(sh, aw1_ref[...], precision=_HI) + ab1_ref[...],
                    0.0)
    logits = jnp.dot(h, aw2_ref[...], precision=_HI) + ab2_ref[...]
    mxl = jnp.max(logits, axis=1, keepdims=True)
    e = jnp.exp(logits - mxl)
    dyn = e / jnp.sum(e, axis=1, keepdims=True)

    @pl.when(pl.program_id(0) == 0)
    def _():
        dyn_ref[...] = jnp.zeros_like(dyn_ref)

    dyn_ref[...] += jnp.sum(dyn, axis=0, keepdims=True)

    hc = jnp.maximum(jnp.dot(sh, cw1_ref[...], precision=_HI) + cb1_ref[...],
                     0.0)
    ac_ref[...] = jax.nn.sigmoid(
        jax.lax.dot_general(cw2_ref[...], hc, (((0,), (1,)), ((), ())),
                            precision=_HI) + cb2_ref[...])


def _prep(gene, cna, mutation, Wg, bg, Wc, bc, Wm, bm,
          attW1, attb1, attW2, attb2, cW1, cb1, cW2, cb2, block=1024):
    n, f = gene.shape
    sd = Wg.shape[1]
    ad = attW1.shape[1]
    bf = jnp.bfloat16
    gmu, gsd, cmu, csd = _col_stats(gene, cna)
    grid = (n // block,)
    row = pl.BlockSpec((block, f), lambda i: (i, 0))
    vec = pl.BlockSpec((1, f), lambda i: (0, 0))
    wsp = pl.BlockSpec((f, sd), lambda i: (0, 0))
    bsp = pl.BlockSpec((1, sd), lambda i: (0, 0))
    cst = lambda shape: pl.BlockSpec(shape, lambda i: tuple(0 for _ in shape))
    return pl.pallas_call(
        _prep_body,
        grid=grid,
        in_specs=[row, row, row, vec, vec, vec, vec,
                  wsp, bsp, wsp, bsp, wsp, bsp,
                  cst((sd, ad)), cst((1, ad)), cst((ad, 3)), cst((1, 3)),
                  cst((sd, ad)), cst((1, ad)), cst((ad, 1)), cst((1, 1))],
        out_specs=[row, row, row,
                   pl.BlockSpec((block, 1), lambda i: (i, 0)),
                   pl.BlockSpec((block, 1), lambda i: (i, 0)),
                   pl.BlockSpec((1, 3), lambda i: (0, 0)),
                   pl.BlockSpec((1, block), lambda i: (0, i))],
        out_shape=[jax.ShapeDtypeStruct((n, f), bf),
                   jax.ShapeDtypeStruct((n, f), bf),
                   jax.ShapeDtypeStruct((n, f), bf),
                   jax.ShapeDtypeStruct((n, 1), jnp.float32),
                   jax.ShapeDtypeStruct((n, 1), jnp.float32),
                   jax.ShapeDtypeStruct((1, 3), jnp.float32),
                   jax.ShapeDtypeStruct((1, n), jnp.float32)],
    )(gene, cna, mutation, gmu, gsd, cmu, csd,
      Wg, bg.reshape(1, -1), Wc, bc.reshape(1, -1), Wm, bm.reshape(1, -1),
      attW1, attb1.reshape(1, -1), attW2, attb2.reshape(1, -1),
      cW1, cb1.reshape(1, -1), cW2, cb2.reshape(1, -1))


def _kernels_body(ghr_ref, ghc_ref, chr_ref, chc_ref,
                  mhr_ref, mhc_ref, sqr_ref, sqc_ref, msr_ref, msc_ref,
                  kg_ref, kc_ref, km_ref, ng_ref, nc_ref, nm_ref):
    i = pl.program_id(0)
    j = pl.program_id(1)

    @pl.when((i == 0) & (j == 0))
    def _():
        ng_ref[...] = jnp.zeros_like(ng_ref)
        nc_ref[...] = jnp.zeros_like(nc_ref)
        nm_ref[...] = jnp.zeros_like(nm_ref)

    gram_g = jax.lax.dot_general(ghr_ref[...], ghc_ref[...], _NT,
                                 preferred_element_type=jnp.float32)
    d2 = jnp.maximum(sqr_ref[...] + sqc_ref[...].T - 2.0 * gram_g, 0.0)
    kg = jnp.exp(d2 * (-1.0 / (2.0 * _SIGMA * _SIGMA)))
    kg_ref[...] = kg.astype(jnp.bfloat16)

    p = jax.lax.dot_general(chr_ref[...], chc_ref[...], _NT,
                            preferred_element_type=jnp.float32) + 1.0
    kc = p * p * p
    kc_ref[...] = kc.astype(jnp.bfloat16)

    inter = jax.lax.dot_general(mhr_ref[...], mhc_ref[...], _NT,
                                preferred_element_type=jnp.float32)
    km = inter / (msr_ref[...] + msc_ref[...].T - inter + 1e-8)
    km_ref[...] = km.astype(jnp.bfloat16)

    ng_ref[...] += jnp.sum(kg * kg, axis=0, keepdims=True)
    nc_ref[...] += jnp.sum(kc * kc, axis=0, keepdims=True)
    nm_ref[...] += jnp.sum(km * km, axis=0, keepdims=True)


def _cell_kernels(gh, ch, mh, sq, ms, tile=1024):
    n, f = gh.shape
    grid = (n // tile, n // tile)
    row = pl.BlockSpec((tile, f), lambda i, j: (i, 0))
    col = pl.BlockSpec((tile, f), lambda i, j: (j, 0))
    vrow = pl.BlockSpec((tile, 1), lambda i, j: (i, 0))
    vcol = pl.BlockSpec((tile, 1), lambda i, j: (j, 0))
    out = pl.BlockSpec((tile, tile), lambda i, j: (i, j))
    nsp = pl.BlockSpec((1, tile), lambda i, j: (0, 0))
    bf = jnp.bfloat16
    return pl.pallas_call(
        _kernels_body,
        grid=grid,
        in_specs=[row, col, row, col, row, col,
                  vrow, vcol, vrow, vcol],
        out_specs=[out, out, out, nsp, nsp, nsp],
        out_shape=[jax.ShapeDtypeStruct((n, n), bf),
                   jax.ShapeDtypeStruct((n, n), bf),
                   jax.ShapeDtypeStruct((n, n), bf),
                   jax.ShapeDtypeStruct((1, tile), jnp.float32),
                   jax.ShapeDtypeStruct((1, tile), jnp.float32),
                   jax.ShapeDtypeStruct((1, tile), jnp.float32)],
    )(gh, gh, ch, ch, mh, mh, sq, sq, ms, ms)


def _topk_threshold(weighted, k):
    neg = jnp.float32(-jnp.inf)
    mx = jnp.max(weighted, axis=1, keepdims=True)
    for _ in range(k - 1):
        mx = jnp.max(jnp.where(weighted < mx, weighted, neg),
                     axis=1, keepdims=True)
    return mx


def _fuse_body(w3_ref, kg_ref, kc_ref, km_ref, att_ref, kcw_ref, kcb_ref,
               out_ref):
    fused = (w3_ref[0] * kg_ref[...].astype(jnp.float32)
             + w3_ref[1] * kc_ref[...].astype(jnp.float32)
             + w3_ref[2] * km_ref[...].astype(jnp.float32))
    weighted = fused * att_ref[...]
    thr = _topk_threshold(weighted, _TOPK)
    filt = jnp.where(weighted >= thr, fused, 0.0)
    out_ref[...] = _dot3(filt, kcw_ref[...], _NN) + kcb_ref[...]


def _fuse_topk(w3, kg, kc, km, att_row, kcW, kcb, block=256):
    n = kg.shape[0]
    sd = kcW.shape[1]
    grid = (n // block,)
    ksp = pl.BlockSpec((block, n), lambda i: (i, 0))
    return pl.pallas_call(
        _fuse_body,
        grid=grid,
        in_specs=[pl.BlockSpec(memory_space=pltpu.SMEM),
                  ksp, ksp, ksp,
                  pl.BlockSpec((1, n), lambda i: (0, 0)),
                  pl.BlockSpec((n, sd), lambda i: (0, 0)),
                  pl.BlockSpec((1, sd), lambda i: (0, 0))],
        out_specs=pl.BlockSpec((block, sd), lambda i: (i, 0)),
        out_shape=jax.ShapeDtypeStruct((n, sd), jnp.float32),
    )(w3, kg, kc, km, att_row, kcW, kcb.reshape(1, -1))


def _drug_body(fd_ref, wd_ref, bd_ref, dw1_ref, db1_ref, dw2_ref, db2_ref,
               kdw_ref, kdb_ref, out_ref):
    fd = fd_ref[...]
    de = _dot1(fd, wd_ref[...], _NN) + bd_ref[...]
    hd = jnp.maximum(jnp.dot(de, dw1_ref[...], precision=_HI) + db1_ref[...],
                     0.0)
    att = jax.nn.sigmoid(
        jax.lax.dot_general(dw2_ref[...], hd, (((0,), (1,)), ((), ())),
                            precision=_HI) + db2_ref[...])
    inter = _dot1(fd, fd, _NT)
    s = jnp.sum(fd, axis=1, keepdims=True)
    jac = inter / (s + s.T - inter + 1e-8)
    weighted = jac * att
    thr = _topk_threshold(weighted, _TOPK)
    filt = jnp.where(weighted >= thr, jac, 0.0)
    d = jnp.sum(filt, axis=1, keepdims=True)
    dinv = jax.lax.rsqrt(d + 1e-8)
    fn = filt * dinv * dinv.T
    out_ref[...] = _dot3(fn, kdw_ref[...], _NN) + kdb_ref[...]


def _drug_branch(feature_drug, Wd, bd, dW1, db1, dW2, db2, kdW, kdb):
    nd = feature_drug.shape[0]
    sd = kdW.shape[1]
    return pl.pallas_call(
        _drug_body,
        out_shape=jax.ShapeDtypeStruct((nd, sd), jnp.float32),
    )(feature_drug, Wd, bd.reshape(1, -1), dW1, db1.reshape(1, -1),
      dW2, db2.reshape(1, -1), kdW, kdb.reshape(1, -1))


@jax.jit
def kernel(gene, cna, mutation, feature_drug, Wg, bg, Wc, bc, Wm, bm, Wd, bd,
           weights3, attW1, attb1, attW2, attb2, cW1, cb1, cW2, cb2,
           dW1, db1, dW2, db2, kcW, kcb, kdW, kdb):
    gh, ch, mh, sq, ms, dyn_sum, att_cell = _prep(
        gene, cna, mutation, Wg, bg, Wc, bc, Wm, bm,
        attW1, attb1, attW2, attb2, cW1, cb1, cW2, cb2)
    kg, kc, km, ng, nc, nm = _cell_kernels(gh, ch, mh, sq, ms)

    n_cell = gene.shape[0]
    stat = jax.nn.softmax(weights3)
    dyn_mean = dyn_sum[0] / n_cell
    w = 0.5 * stat + 0.5 * dyn_mean
    norms = jnp.sqrt(jnp.stack([jnp.sum(ng), jnp.sum(nc), jnp.sum(nm)]))
    w3 = w / (norms + 1e-8)

    cell_feat = _fuse_topk(w3, kg, kc, km, att_cell, kcW, kcb)
    drug_feat = _drug_branch(feature_drug, Wd, bd, dW1, db1, dW2, db2,
                             kdW, kdb)
    return jnp.concatenate([cell_feat, drug_feat], axis=0)
